# Initial kernel scaffold; baseline (speedup 1.0000x reference)
import jax
import jax.numpy as jnp
from jax.experimental import pallas as pl


def kernel(x, embeddings):
    raise NotImplementedError("write your pallas kernel here")


# speedup vs baseline: 1.4198x; 1.4198x over previous
import functools

import jax
import jax.numpy as jnp
from jax import lax
from jax.experimental import pallas as pl
from jax.experimental.pallas import tpu as pltpu
from jax.experimental.pallas import tpu_sc as plsc

NE = 8192
ED = 256
ROWS_BLOCK = 256
WINDOWS = ((0, 2816), (2816, 5632), (5632, 8192))


def _argmin_body(x_ref, emb_ref, xsq_ref, esq_ref, out_ref):
    x = x_ref[...]
    emb = emb_ref[...]
    x_sq = xsq_ref[...]
    e_sq = esq_ref[...]
    sim = jnp.dot(x, emb, preferred_element_type=jnp.float32)
    d = (x_sq + e_sq) - (sim * 2.0)

    acc_v = jnp.full((x.shape[0],), jnp.inf, jnp.float32)
    acc_i = jnp.zeros((x.shape[0],), jnp.int32)
    for lo, hi in WINDOWS:
        blk = d[:, lo:hi]
        v = jnp.min(blk, axis=1)
        it = lax.broadcasted_iota(jnp.int32, blk.shape, 1) + lo
        vi = jnp.min(jnp.where(blk <= v[:, None], it, NE), axis=1)
        steal = v < acc_v
        acc_v = jnp.where(steal, v.astype(jnp.bfloat16).astype(jnp.float32),
                          acc_v)
        acc_i = jnp.where(steal, vi, acc_i)
    out_ref[...] = acc_i


def _compute_indices(flat, embeddings, x_sq, e_sq):
    b = flat.shape[0]
    grid = b // ROWS_BLOCK
    return pl.pallas_call(
        _argmin_body,
        grid=(grid,),
        in_specs=[
            pl.BlockSpec((ROWS_BLOCK, ED), lambda i: (i, 0)),
            pl.BlockSpec((ED, NE), lambda i: (0, 0)),
            pl.BlockSpec((ROWS_BLOCK, 1), lambda i: (i, 0)),
            pl.BlockSpec((1, NE), lambda i: (0, 0)),
        ],
        out_specs=pl.BlockSpec((ROWS_BLOCK,), lambda i: (i,)),
        out_shape=jax.ShapeDtypeStruct((b,), jnp.int32),
    )(flat, embeddings, x_sq, e_sq)


def _gather_rows(table, idx):
    info = plsc.get_sparse_core_info()
    nc, ns = info.num_cores, info.num_subcores
    nw = nc * ns
    b = idx.shape[0]
    b_per_w = b // nw
    ch = 128
    nch = b_per_w // ch
    mesh = plsc.VectorSubcoreMesh(core_axis_name="c", subcore_axis_name="s")

    @functools.partial(
        pl.kernel,
        mesh=mesh,
        out_type=jax.ShapeDtypeStruct((b, ED), jnp.float32),
        scratch_types=[
            pltpu.VMEM((b_per_w,), jnp.int32),
            pltpu.VMEM((ch, ED), jnp.float32),
            pltpu.VMEM((ch, ED), jnp.float32),
            pltpu.SemaphoreType.DMA,
            pltpu.SemaphoreType.DMA,
        ],
    )
    def gk(table_hbm, idx_hbm, out_hbm, idx_v, buf0, buf1, s0, s1):
        wid = lax.axis_index("s") * nc + lax.axis_index("c")
        base = wid * b_per_w
        pltpu.sync_copy(idx_hbm.at[pl.ds(base, b_per_w)], idx_v)
        bufs = (buf0, buf1)
        sems = (s0, s1)

        def start(c):
            return pltpu.async_copy(
                table_hbm.at[idx_v.at[pl.ds(c * ch, ch)]],
                bufs[c % 2],
                sems[c % 2],
            )

        cur = start(0)
        for c in range(nch):
            nxt = start(c + 1) if c + 1 < nch else None
            cur.wait()
            pltpu.sync_copy(bufs[c % 2], out_hbm.at[pl.ds(base + c * ch, ch)])
            cur = nxt

    return gk(table, idx)


def kernel(x, embeddings):
    input_shape = x.shape
    flat = x.reshape(-1, ED)
    x_sq = jnp.sum(flat ** 2, axis=1, keepdims=True)
    e_sq = jnp.sum(embeddings ** 2, axis=0, keepdims=True)
    indices = _compute_indices(flat, embeddings, x_sq, e_sq)
    table = embeddings.T
    quant = _gather_rows(table, indices)
    return quant.reshape(input_shape)


# speedup vs baseline: 1.7969x; 1.2656x over previous
import functools

import jax
import jax.numpy as jnp
from jax import lax
from jax.experimental import pallas as pl
from jax.experimental.pallas import tpu as pltpu
from jax.experimental.pallas import tpu_sc as plsc

NE = 8192
ED = 256
ROWS_BLOCK = 512
WINDOWS = ((0, 2816), (2816, 5632), (5632, 8192))


def _argmin_body(x_ref, emb_ref, xsq_ref, esq_ref, iota_ref, out_ref):
    x = x_ref[...]
    emb = emb_ref[...]
    x_sq = xsq_ref[...]
    e_sq = esq_ref[...]
    iota = iota_ref[...]
    sim = jnp.dot(x, emb, preferred_element_type=jnp.float32)
    d = (x_sq + e_sq) - (sim * 2.0)

    acc_v = jnp.full((x.shape[0],), jnp.inf, jnp.float32)
    acc_i = jnp.full((x.shape[0],), jnp.inf, jnp.float32)
    for lo, hi in WINDOWS:
        blk = d[:, lo:hi]
        v = jnp.min(blk, axis=1)
        it = jnp.broadcast_to(iota[:, lo:hi], blk.shape)
        vi = jnp.min(jnp.where(blk <= v[:, None], it, jnp.inf), axis=1)
        steal = v < acc_v
        acc_v = jnp.where(steal, v.astype(jnp.bfloat16).astype(jnp.float32),
                          acc_v)
        acc_i = jnp.where(steal, vi, acc_i)
    out_ref[...] = acc_i.astype(jnp.int32)


def _compute_indices(flat, embeddings, x_sq, e_sq, iota):
    b = flat.shape[0]
    grid = b // ROWS_BLOCK
    return pl.pallas_call(
        _argmin_body,
        grid=(grid,),
        in_specs=[
            pl.BlockSpec((ROWS_BLOCK, ED), lambda i: (i, 0)),
            pl.BlockSpec((ED, NE), lambda i: (0, 0)),
            pl.BlockSpec((ROWS_BLOCK, 1), lambda i: (i, 0)),
            pl.BlockSpec((1, NE), lambda i: (0, 0)),
            pl.BlockSpec((1, NE), lambda i: (0, 0)),
        ],
        out_specs=pl.BlockSpec((ROWS_BLOCK,), lambda i: (i,)),
        out_shape=jax.ShapeDtypeStruct((b,), jnp.int32),
    )(flat, embeddings, x_sq, e_sq, iota)


def _gather_rows(table, idx):
    info = plsc.get_sparse_core_info()
    nc, ns = info.num_cores, info.num_subcores
    nw = nc * ns
    b = idx.shape[0]
    b_per_w = b // nw
    ch = 128
    nch = b_per_w // ch
    mesh = plsc.VectorSubcoreMesh(core_axis_name="c", subcore_axis_name="s")

    @functools.partial(
        pl.kernel,
        mesh=mesh,
        out_type=jax.ShapeDtypeStruct((b, ED), jnp.float32),
        scratch_types=[
            pltpu.VMEM((b_per_w,), jnp.int32),
            pltpu.VMEM((ch, ED), jnp.float32),
            pltpu.VMEM((ch, ED), jnp.float32),
            pltpu.SemaphoreType.DMA,
            pltpu.SemaphoreType.DMA,
        ],
    )
    def gk(table_hbm, idx_hbm, out_hbm, idx_v, buf0, buf1, s0, s1):
        wid = lax.axis_index("s") * nc + lax.axis_index("c")
        base = wid * b_per_w
        pltpu.sync_copy(idx_hbm.at[pl.ds(base, b_per_w)], idx_v)
        bufs = (buf0, buf1)
        sems = (s0, s1)

        def start(c):
            return pltpu.async_copy(
                table_hbm.at[idx_v.at[pl.ds(c * ch, ch)]],
                bufs[c % 2],
                sems[c % 2],
            )

        cur = start(0)
        for c in range(nch):
            nxt = start(c + 1) if c + 1 < nch else None
            cur.wait()
            pltpu.sync_copy(bufs[c % 2], out_hbm.at[pl.ds(base + c * ch, ch)])
            cur = nxt

    return gk(table, idx)


def kernel(x, embeddings):
    input_shape = x.shape
    flat = x.reshape(-1, ED)
    x_sq = jnp.sum(flat ** 2, axis=1, keepdims=True)
    e_sq = jnp.sum(embeddings ** 2, axis=0, keepdims=True)
    iota = jnp.arange(NE, dtype=jnp.float32).reshape(1, NE)
    indices = _compute_indices(flat, embeddings, x_sq, e_sq, iota)
    table = embeddings.T
    quant = _gather_rows(table, indices)
    return quant.reshape(input_shape)
